# Initial kernel scaffold; baseline (speedup 1.0000x reference)
import jax
import jax.numpy as jnp
from jax.experimental import pallas as pl


def kernel(x_num, x_cat, emb_tables, W1, b1, W2, b2, W3, b3, W4, b4):
    raise NotImplementedError("write your pallas kernel here")


# speedup vs baseline: 7.5788x; 7.5788x over previous
import functools

import jax
import jax.numpy as jnp
from jax import lax
from jax.experimental import pallas as pl
from jax.experimental.pallas import tpu as pltpu
from jax.experimental.pallas import tpu_sc as plsc

NUM_FIELDS = 26
VOCAB = 1000
EMB_DIM = 50
NUM_NUMERIC = 13
BATCH = 16384
H1 = 128

NC = 2
NS = 16
NW = NC * NS
BPW = BATCH // NW
RPC = 4
GROUP = RPC * NUM_FIELDS
NCHUNK = BPW // RPC
LANES = 16


def _fold_body(e_ref, w_ref, o_ref):
    o_ref[0] = jnp.dot(e_ref[0], w_ref[0], preferred_element_type=jnp.float32)


def _fold_tables(emb_tables, w1_emb):
    return pl.pallas_call(
        _fold_body,
        grid=(NUM_FIELDS,),
        in_specs=[
            pl.BlockSpec((1, VOCAB, EMB_DIM), lambda f: (f, 0, 0)),
            pl.BlockSpec((1, EMB_DIM, H1), lambda f: (f, 0, 0)),
        ],
        out_specs=pl.BlockSpec((1, VOCAB, H1), lambda f: (f, 0, 0)),
        out_shape=jax.ShapeDtypeStruct((NUM_FIELDS, VOCAB, H1), jnp.float32),
    )(emb_tables, w1_emb)


def _gather_sum_body(t_hbm, idx_hbm, dst_hbm, zeros_hbm, out_hbm,
                     idx_v, dst_v, buf0, buf1, acc, sem0, sem1):
    cid = lax.axis_index("c")
    sid = lax.axis_index("s")
    wid = sid * NC + cid
    pltpu.sync_copy(idx_hbm.at[wid], idx_v)
    pltpu.sync_copy(dst_hbm.at[sid], dst_v)
    my_acc = acc.at[pl.ds(sid * BPW, BPW)]
    pltpu.sync_copy(zeros_hbm, my_acc)

    pltpu.async_copy(t_hbm.at[idx_v.at[0]], buf0, sem0)

    def body(c2, _):
        c = c2 * 2
        pltpu.async_copy(t_hbm.at[idx_v.at[c + 1]], buf1, sem1)
        pltpu.make_async_copy(t_hbm.at[idx_v.at[c]], buf0, sem0).wait()
        pltpu.sync_copy(buf0, acc.at[dst_v.at[c]], add=True)
        nxt = lax.rem(c + 2, NCHUNK)
        pltpu.async_copy(t_hbm.at[idx_v.at[nxt]], buf0, sem0)
        pltpu.make_async_copy(t_hbm.at[idx_v.at[c + 1]], buf1, sem1).wait()
        pltpu.sync_copy(buf1, acc.at[dst_v.at[c + 1]], add=True)
        return 0

    lax.fori_loop(0, NCHUNK // 2, body, 0)
    pltpu.make_async_copy(t_hbm.at[idx_v.at[0]], buf0, sem0).wait()

    pltpu.sync_copy(my_acc, out_hbm.at[pl.ds(wid * BPW, BPW)])


def _gather_sum(t_flat, idx3, dst3, zeros):
    mesh = plsc.VectorSubcoreMesh(core_axis_name="c", subcore_axis_name="s",
                                  num_cores=NC, num_subcores=NS)
    f = functools.partial(
        pl.kernel,
        out_type=jax.ShapeDtypeStruct((BATCH, H1), jnp.float32),
        mesh=mesh,
        scratch_types=[
            pltpu.VMEM((NCHUNK, GROUP), jnp.int32),
            pltpu.VMEM((NCHUNK, GROUP), jnp.int32),
            pltpu.VMEM((GROUP, H1), jnp.float32),
            pltpu.VMEM((GROUP, H1), jnp.float32),
            pltpu.VMEM_SHARED((NS * BPW, H1), jnp.float32),
            pltpu.SemaphoreType.DMA,
            pltpu.SemaphoreType.DMA,
        ],
    )(_gather_sum_body)
    return f(t_flat, idx3, dst3, zeros)


def _mlp_body(g_ref, xn_ref, w1n_ref, b1_ref, w2_ref, b2_ref, w3_ref, b3_ref,
              w4_ref, b4_ref, o_ref):
    h = g_ref[...] + jnp.dot(xn_ref[...], w1n_ref[...],
                             preferred_element_type=jnp.float32)
    h = jax.nn.relu(h + b1_ref[...])
    h = jax.nn.relu(jnp.dot(h, w2_ref[...], preferred_element_type=jnp.float32)
                    + b2_ref[...])
    h = jax.nn.relu(jnp.dot(h, w3_ref[...], preferred_element_type=jnp.float32)
                    + b3_ref[...])
    o_ref[...] = (jnp.dot(h, w4_ref[...], preferred_element_type=jnp.float32)
                  + b4_ref[...])


def _mlp_tail(g, x_num, w1n, b1, w2, b2, w3, b3, w4, b4):
    BB = 2048
    full = lambda *shape: pl.BlockSpec(shape, lambda i: (0,) * len(shape))
    return pl.pallas_call(
        _mlp_body,
        grid=(BATCH // BB,),
        in_specs=[
            pl.BlockSpec((BB, H1), lambda i: (i, 0)),
            pl.BlockSpec((BB, NUM_NUMERIC), lambda i: (i, 0)),
            full(NUM_NUMERIC, H1),
            full(H1),
            full(H1, 64),
            full(64),
            full(64, 32),
            full(32),
            full(32, 1),
            full(1),
        ],
        out_specs=pl.BlockSpec((BB, 1), lambda i: (i, 0)),
        out_shape=jax.ShapeDtypeStruct((BATCH, 1), jnp.float32),
    )(g, x_num, w1n, b1, w2, b2, w3, b3, w4, b4)


def kernel(x_num, x_cat, emb_tables, W1, b1, W2, b2, W3, b3, W4, b4):
    w1_num = W1[:NUM_NUMERIC]
    w1_emb = W1[NUM_NUMERIC:].reshape(NUM_FIELDS, EMB_DIM, H1)

    t = _fold_tables(emb_tables, w1_emb)
    t_flat = t.reshape(NUM_FIELDS * VOCAB, H1)

    idx = x_cat + (jnp.arange(NUM_FIELDS, dtype=jnp.int32) * VOCAB)[None, :]
    idx3 = idx.reshape(NW, NCHUNK, GROUP)
    pat = jnp.arange(GROUP, dtype=jnp.int32) // NUM_FIELDS
    dst3 = (jnp.arange(NS, dtype=jnp.int32)[:, None, None] * BPW
            + jnp.arange(NCHUNK, dtype=jnp.int32)[None, :, None] * RPC
            + pat[None, None, :])
    zeros = jnp.zeros((BPW, H1), jnp.float32)

    g = _gather_sum(t_flat, idx3, dst3, zeros)

    out = _mlp_tail(g, x_num, w1_num, b1, W2, b2, W3, b3, W4, b4)
    return out[:, 0]
